# baseline (speedup 1.0000x reference)
import dataclasses
import functools
import math

import jax
import jax.numpy as jnp
from jax import lax
from jax.experimental import pallas as pl
from jax.experimental.pallas import tpu as pltpu
from jax.experimental.pallas import tpu_sc as plsc

_CLASSES = 32000
_SMOOTHING = 0.2
_CONF = 1.0 - _SMOOTHING
_EPS = _SMOOTHING / (_CLASSES - 2)
_C0 = _CONF * math.log(_CONF) + _SMOOTHING * math.log(_EPS)

_NT = 3328
_BR = 832
_BC = 6400

_SC_W = 32
_SC_BANDS = 3
_SC_CHUNK = 3200
_SC_NCH = _CLASSES // _SC_CHUNK


def _tc_body(pred_ref, targ_ref, out_ref):
    j = pl.program_id(1)
    block = pred_ref[...]
    targ = targ_ref[...]
    w = (targ != 0).astype(jnp.float32)
    cols = jax.lax.broadcasted_iota(jnp.int32, (_BR, _BC), 1) + j * _BC
    is_t = (cols == targ).astype(jnp.float32)
    coeff = (_EPS - _CONF) * is_t - _EPS
    rowsum = jnp.sum(block * coeff, axis=1, keepdims=True)
    partial = jnp.sum(rowsum * w)

    @pl.when(j == 0)
    def _():
        p0 = block[:, 0:1]
        out_ref[0, 0, 0] = partial + jnp.sum((_C0 + _EPS * p0) * w)

    @pl.when(j != 0)
    def _():
        out_ref[0, 0, 0] += partial


def _tree_rowsum(buf, r):
    vals = [buf[r, pl.ds(k, 16)] for k in range(0, _SC_CHUNK, 16)]
    while len(vals) > 1:
        nxt = [vals[i] + vals[i + 1] for i in range(0, len(vals) - 1, 2)]
        if len(vals) % 2:
            nxt.append(vals[-1])
        vals = nxt
    return vals[0]


_SC_MESH = plsc.VectorSubcoreMesh(core_axis_name="c", subcore_axis_name="s")

_SC_PARAMS = pltpu.CompilerParams()
if "needs_layout_passes" in pltpu.CompilerParams.__dataclass_fields__:
    _SC_PARAMS = dataclasses.replace(_SC_PARAMS, needs_layout_passes=False)


@functools.partial(
    pl.kernel,
    mesh=_SC_MESH,
    compiler_params=_SC_PARAMS,
    out_type=jax.ShapeDtypeStruct((_SC_W * 16,), jnp.float32),
    scratch_types=[
        pltpu.VMEM((8, _SC_CHUNK), jnp.float32),
        pltpu.VMEM((8, _SC_CHUNK), jnp.float32),
        pltpu.VMEM((8, 16), jnp.float32),
        pltpu.VMEM((16,), jnp.float32),
        pltpu.VMEM((16,), jnp.float32),
        pltpu.VMEM((16,), jnp.float32),
        pltpu.VMEM((32,), jnp.int32),
        pltpu.SemaphoreType.DMA,
        pltpu.SemaphoreType.DMA,
    ],
)
def _sc_tail(pred_hbm, targ_hbm, out_hbm,
             buf0, buf1, accb, ptb, p0b, tot, targ_v, sem0, sem1):
    w = lax.axis_index("c") * 16 + lax.axis_index("s")
    row0 = _NT + w * 24
    pltpu.sync_copy(targ_hbm.at[pl.ds(row0, 16)], targ_v.at[pl.ds(0, 16)])
    pltpu.sync_copy(targ_hbm.at[pl.ds(row0 + 8, 16)], targ_v.at[pl.ds(16, 16)])
    tot[...] = jnp.zeros((16,), jnp.float32)
    lane = lax.iota(jnp.int32, 16)
    lane8 = lane & 7
    low = lane < 8

    for band in range(_SC_BANDS):
        r0 = row0 + band * 8
        tvec = plsc.load_gather(targ_v, [(0, 8, 24)[band] + lane8])
        for r in range(8):
            accb[r, ...] = jnp.zeros((16,), jnp.float32)
        ptb[...] = jnp.zeros((16,), jnp.float32)
        p0b[...] = jnp.zeros((16,), jnp.float32)

        def _fire(c, buf, sem):
            pltpu.async_copy(
                pred_hbm.at[pl.ds(r0, 8), pl.ds(c * _SC_CHUNK, _SC_CHUNK)],
                buf, sem)

        def _wait(c, buf, sem):
            pltpu.make_async_copy(
                pred_hbm.at[pl.ds(r0, 8), pl.ds(c * _SC_CHUNK, _SC_CHUNK)],
                buf, sem).wait()

        def _process(buf, c):
            c0 = c * _SC_CHUNK

            @pl.loop(0, 8)
            def _(r):
                accb[r, ...] = accb[r, ...] + _tree_rowsum(buf, r)

            rel = tvec - c0
            inr = (rel >= 0) & (rel < _SC_CHUNK) & low
            col = jnp.minimum(jnp.maximum(rel, 0), _SC_CHUNK - 1)
            g = plsc.load_gather(buf, [lane8, col])
            ptb[...] = jnp.where(inr, g, ptb[...])

            @pl.when(c0 == 0)
            def _():
                g0 = plsc.load_gather(buf, [lane8, jnp.zeros_like(lane)])
                p0b[...] = jnp.where(low, g0, p0b[...])

        _fire(0, buf0, sem0)
        _fire(1, buf1, sem1)

        @pl.loop(0, _SC_NCH // 2)
        def _(cp):
            c = 2 * cp
            _wait(c, buf0, sem0)
            _process(buf0, c)

            @pl.when(cp < _SC_NCH // 2 - 1)
            def _():
                _fire(c + 2, buf0, sem0)

            _wait(c + 1, buf1, sem1)
            _process(buf1, c + 1)

            @pl.when(cp < _SC_NCH // 2 - 1)
            def _():
                _fire(c + 3, buf1, sem1)

        rv = jnp.zeros((16,), jnp.float32)
        for r in range(8):
            rv = jnp.where(lane == r, jnp.sum(accb[r, ...]), rv)
        valid = (tvec != 0) & low
        contrib = (_C0 + (_EPS - _CONF) * ptb[...] + _EPS * p0b[...]
                   - _EPS * rv)
        tot[...] = tot[...] + jnp.where(valid, contrib, 0.0)

    pltpu.sync_copy(tot, out_hbm.at[pl.ds(w * 16, 16)])


def kernel(pred, target):
    n, c = pred.shape
    targ1d = target.astype(jnp.int32)
    targ2d = targ1d.reshape(n, 1)
    grid = (_NT // _BR, c // _BC)
    tc_partials = pl.pallas_call(
        _tc_body,
        grid=grid,
        in_specs=[
            pl.BlockSpec((_BR, _BC), lambda i, j: (i, j)),
            pl.BlockSpec((_BR, 1), lambda i, j: (i, 0)),
        ],
        out_specs=pl.BlockSpec((1, 1, 1), lambda i, j: (i, 0, 0),
                               memory_space=pltpu.SMEM),
        out_shape=jax.ShapeDtypeStruct((grid[0], 1, 1), jnp.float32),
        compiler_params=pltpu.CompilerParams(
            dimension_semantics=("parallel", "arbitrary")),
    )(pred, targ2d)
    sc_partials = _sc_tail(pred, targ1d)
    return jnp.sum(tc_partials) + jnp.sum(sc_partials)


# speedup vs baseline: 1.1266x; 1.1266x over previous
import math

import jax
import jax.numpy as jnp
from jax.experimental import pallas as pl
from jax.experimental.pallas import tpu as pltpu

_CLASSES = 32000
_SMOOTHING = 0.2
_CONF = 1.0 - _SMOOTHING
_EPS = _SMOOTHING / (_CLASSES - 2)
_C0 = _CONF * math.log(_CONF) + _SMOOTHING * math.log(_EPS)

_BR = 1024
_BC = 6400


def _body(pred_ref, targ_ref, out_ref):
    j = pl.program_id(1)
    block = pred_ref[...]
    targ = targ_ref[...]
    w = (targ != 0).astype(jnp.float32)
    cols = jax.lax.broadcasted_iota(jnp.int32, (_BR, _BC), 1) + j * _BC
    is_t = (cols == targ).astype(jnp.float32)
    coeff = (_EPS - _CONF) * is_t - _EPS
    rowsum = jnp.sum(block * coeff, axis=1, keepdims=True)
    partial = jnp.sum(rowsum * w)

    @pl.when(j == 0)
    def _():
        p0 = block[:, 0:1]
        out_ref[0, 0, 0] = partial + jnp.sum((_C0 + _EPS * p0) * w)

    @pl.when(j != 0)
    def _():
        out_ref[0, 0, 0] += partial


def kernel(pred, target):
    n, c = pred.shape
    targ2d = target.reshape(n, 1).astype(jnp.int32)
    grid = (n // _BR, c // _BC)
    partials = pl.pallas_call(
        _body,
        grid=grid,
        in_specs=[
            pl.BlockSpec((_BR, _BC), lambda i, j: (i, j)),
            pl.BlockSpec((_BR, 1), lambda i, j: (i, 0)),
        ],
        out_specs=pl.BlockSpec((1, 1, 1), lambda i, j: (i, 0, 0),
                               memory_space=pltpu.SMEM),
        out_shape=jax.ShapeDtypeStruct((grid[0], 1, 1), jnp.float32),
        compiler_params=pltpu.CompilerParams(
            dimension_semantics=("parallel", "arbitrary")),
    )(pred, targ2d)
    return jnp.sum(partials)
